# baseline (speedup 1.0000x reference)
import jax
import jax.numpy as jnp
from jax.experimental import pallas as pl

_B = 16384
_IN = 768
_H = 512
_E = 64
_K = 1024
_L = 3
_BB = 2048


_CH = 1024


def _chunk_loss(x, w1, b1, w2, b2, w3, b3,
                dw1, db1, dw2, db2, dw3, db3, cb, cbt):
    f32 = jnp.float32
    bf16 = jnp.bfloat16

    h = jnp.maximum(jnp.dot(x.astype(bf16), w1[...], preferred_element_type=f32) + b1[...], 0.0).astype(bf16)
    h = jnp.maximum(jnp.dot(h, w2[...], preferred_element_type=f32) + b2[...], 0.0).astype(bf16)
    z = jnp.dot(h, w3[...], preferred_element_type=f32) + b3[...]

    ones_col = jnp.ones((_CH, 1), bf16)
    r = z
    e_sum = jnp.zeros_like(z)
    q_sum = f32(0.0)
    for l in range(_L):
        cbtl = cbt[l]
        cbf = cbtl.astype(f32)
        hn = -0.5 * jnp.sum(cbf * cbf, axis=0)
        cbta = jnp.concatenate([cbtl, hn[None, :].astype(bf16)], axis=0)
        ra = jnp.concatenate([r.astype(bf16), ones_col], axis=1)
        g = jnp.dot(ra, cbta, preferred_element_type=f32)
        mx = jnp.max(g, axis=1, keepdims=True)
        onehot = (g >= mx).astype(bf16)
        e = jnp.dot(onehot, cb[l], preferred_element_type=f32)
        r = r - e
        q_sum = q_sum + jnp.sum(r * r)
        e_sum = e_sum + e

    h = jnp.maximum(jnp.dot(e_sum.astype(bf16), dw1[...], preferred_element_type=f32) + db1[...], 0.0).astype(bf16)
    h = jnp.maximum(jnp.dot(h, dw2[...], preferred_element_type=f32) + db2[...], 0.0).astype(bf16)
    x_hat = jnp.dot(h, dw3[...], preferred_element_type=f32) + db3[...]

    diff = x_hat - x
    return jnp.sum(diff * diff) + 1.25 * q_sum


def _fused_kernel(x_ref, w1, b1, w2, b2, w3, b3,
                  dw1, db1, dw2, db2, dw3, db3, cb, cbt, out_ref):
    i = pl.program_id(0)
    args = (w1, b1, w2, b2, w3, b3, dw1, db1, dw2, db2, dw3, db3, cb, cbt)
    part = _chunk_loss(x_ref[0:_CH, :], *args)
    part = part + _chunk_loss(x_ref[_CH:_BB, :], *args)
    part = part * (1.0 / _B)

    @pl.when(i == 0)
    def _init():
        out_ref[...] = jnp.zeros_like(out_ref)

    out_ref[...] += jnp.reshape(part, (1, 1))


def kernel(x, enc_w1, enc_b1, enc_w2, enc_b2, enc_w3, enc_b3,
           dec_w1, dec_b1, dec_w2, dec_b2, dec_w3, dec_b3, codebooks, t):
    del t
    full = lambda a: pl.BlockSpec(a.shape, lambda i: (0,) * a.ndim)
    bf16 = jnp.bfloat16
    enc_w1 = enc_w1.astype(bf16)
    enc_w2 = enc_w2.astype(bf16)
    enc_w3 = enc_w3.astype(bf16)
    dec_w1 = dec_w1.astype(bf16)
    dec_w2 = dec_w2.astype(bf16)
    dec_w3 = dec_w3.astype(bf16)
    cb = codebooks.astype(bf16)
    cbt = cb.transpose(0, 2, 1)
    b1 = enc_b1.reshape(1, _H).astype(bf16)
    b2 = enc_b2.reshape(1, _H).astype(bf16)
    b3 = enc_b3.reshape(1, _E)
    db1 = dec_b1.reshape(1, _H).astype(bf16)
    db2 = dec_b2.reshape(1, _H).astype(bf16)
    db3 = dec_b3.reshape(1, _IN)
    out = pl.pallas_call(
        _fused_kernel,
        grid=(_B // _BB,),
        in_specs=[
            pl.BlockSpec((_BB, _IN), lambda i: (i, 0)),
            full(enc_w1), full(b1), full(enc_w2), full(b2),
            full(enc_w3), full(b3),
            full(dec_w1), full(db1), full(dec_w2), full(db2),
            full(dec_w3), full(db3),
            full(cb), full(cbt),
        ],
        out_specs=pl.BlockSpec((1, 1), lambda i: (0, 0)),
        out_shape=jax.ShapeDtypeStruct((1, 1), jnp.float32),
    )(x, enc_w1, b1, enc_w2, b2, enc_w3, b3,
      dec_w1, db1, dec_w2, db2, dec_w3, db3, cb, cbt)
    return out[0, 0]


# speedup vs baseline: 1.0563x; 1.0563x over previous
import jax
import jax.numpy as jnp
from jax.experimental import pallas as pl
from jax.experimental.pallas import tpu as pltpu

_B = 16384
_IN = 768
_H = 512
_E = 64
_K = 1024
_L = 3
_BB = 2048
_CH = 1024


def _chunk_loss(x, sw1, b1, sw2, b2, sw3, b3,
                sdw1, db1, sdw2, db2, sdw3, db3, scb, scbta):
    f32 = jnp.float32
    bf16 = jnp.bfloat16

    h = jnp.maximum(jnp.dot(x.astype(bf16), sw1[...], preferred_element_type=f32) + b1[...], 0.0).astype(bf16)
    h = jnp.maximum(jnp.dot(h, sw2[...], preferred_element_type=f32) + b2[...], 0.0).astype(bf16)
    z = jnp.dot(h, sw3[...], preferred_element_type=f32) + b3[...]

    ones_col = jnp.ones((_CH, 1), bf16)
    r = z
    e_sum = jnp.zeros_like(z)
    q_sum = f32(0.0)
    for l in range(_L):
        ra = jnp.concatenate([r.astype(bf16), ones_col], axis=1)
        g = jnp.dot(ra, scbta[l], preferred_element_type=f32)
        mx = jnp.max(g, axis=1, keepdims=True)
        onehot = (g >= mx).astype(bf16)
        e = jnp.dot(onehot, scb[l], preferred_element_type=f32)
        r = r - e
        q_sum = q_sum + jnp.sum(r * r)
        e_sum = e_sum + e

    h = jnp.maximum(jnp.dot(e_sum.astype(bf16), sdw1[...], preferred_element_type=f32) + db1[...], 0.0).astype(bf16)
    h = jnp.maximum(jnp.dot(h, sdw2[...], preferred_element_type=f32) + db2[...], 0.0).astype(bf16)
    x_hat = jnp.dot(h, sdw3[...], preferred_element_type=f32) + db3[...]

    diff = x_hat - x
    return jnp.sum(diff * diff) + 1.25 * q_sum


def _fused_kernel(x_ref, w1, b1, w2, b2, w3, b3,
                  dw1, db1, dw2, db2, dw3, db3, cb, out_ref,
                  sw1, sw2, sw3, sdw1, sdw2, sdw3, scb, scbta):
    i = pl.program_id(0)
    bf16 = jnp.bfloat16

    @pl.when(i == 0)
    def _prep():
        out_ref[...] = jnp.zeros_like(out_ref)
        sw1[...] = w1[...].astype(bf16)
        sw2[...] = w2[...].astype(bf16)
        sw3[...] = w3[...].astype(bf16)
        sdw1[...] = dw1[...].astype(bf16)
        sdw2[...] = dw2[...].astype(bf16)
        sdw3[...] = dw3[...].astype(bf16)
        for l in range(_L):
            cbl = cb[l]
            scb[l] = cbl.astype(bf16)
            cbtf = cbl.T
            hn = -0.5 * jnp.sum(cbtf * cbtf, axis=0)
            scbta[l] = jnp.concatenate(
                [cbtf.astype(bf16), hn[None, :].astype(bf16)], axis=0)

    args = (sw1, b1, sw2, b2, sw3, b3, sdw1, db1, sdw2, db2, sdw3, db3,
            scb, scbta)
    part = _chunk_loss(x_ref[0:_CH, :], *args)
    part = part + _chunk_loss(x_ref[_CH:_BB, :], *args)
    out_ref[...] += jnp.reshape(part * (1.0 / _B), (1, 1))


def kernel(x, enc_w1, enc_b1, enc_w2, enc_b2, enc_w3, enc_b3,
           dec_w1, dec_b1, dec_w2, dec_b2, dec_w3, dec_b3, codebooks, t):
    del t
    full = lambda a: pl.BlockSpec(a.shape, lambda i: (0,) * a.ndim)
    bf16 = jnp.bfloat16
    b1 = enc_b1.reshape(1, _H)
    b2 = enc_b2.reshape(1, _H)
    b3 = enc_b3.reshape(1, _E)
    db1 = dec_b1.reshape(1, _H)
    db2 = dec_b2.reshape(1, _H)
    db3 = dec_b3.reshape(1, _IN)
    out = pl.pallas_call(
        _fused_kernel,
        grid=(_B // _BB,),
        in_specs=[
            pl.BlockSpec((_BB, _IN), lambda i: (i, 0)),
            full(enc_w1), full(b1), full(enc_w2), full(b2),
            full(enc_w3), full(b3),
            full(dec_w1), full(db1), full(dec_w2), full(db2),
            full(dec_w3), full(db3),
            full(codebooks),
        ],
        out_specs=pl.BlockSpec((1, 1), lambda i: (0, 0)),
        out_shape=jax.ShapeDtypeStruct((1, 1), jnp.float32),
        scratch_shapes=[
            pltpu.VMEM((_IN, _H), bf16),
            pltpu.VMEM((_H, _H), bf16),
            pltpu.VMEM((_H, _E), bf16),
            pltpu.VMEM((_E, _H), bf16),
            pltpu.VMEM((_H, _H), bf16),
            pltpu.VMEM((_H, _IN), bf16),
            pltpu.VMEM((_L, _K, _E), bf16),
            pltpu.VMEM((_L, _E + 1, _K), bf16),
        ],
    )(x, enc_w1, b1, enc_w2, b2, enc_w3, b3,
      dec_w1, db1, dec_w2, db2, dec_w3, db3, codebooks)
    return out[0, 0]
